# baseline (speedup 1.0000x reference)
import functools

import jax
import jax.numpy as jnp
from jax import lax
from jax.experimental import pallas as pl
from jax.experimental.pallas import tpu as pltpu
from jax.experimental.pallas import tpu_sc as plsc

_B0, _B1 = 4096, 200
_D = 64
_B_TOTAL = _B0 * _B1
_NC, _NS = 2, 16
_NW = _NC * _NS
_ROWS_PER_W = _B_TOTAL // _NW
_CHUNK = 128
_N_CHUNKS = _ROWS_PER_W // _CHUNK

_mesh = plsc.VectorSubcoreMesh(core_axis_name="c", subcore_axis_name="s")


@functools.partial(
    pl.kernel,
    mesh=_mesh,
    compiler_params=pltpu.CompilerParams(use_tc_tiling_on_sc=False),
    out_type=jax.ShapeDtypeStruct((_B_TOTAL, _D), jnp.float32),
    scratch_types=[
        pltpu.VMEM((_N_CHUNKS, _CHUNK), jnp.int32),
        pltpu.VMEM((_CHUNK, _D), jnp.float32),
        pltpu.SemaphoreType.DMA,
    ],
)
def _gather(idx_hbm, table_hbm, out_hbm, idx_v, rows_v, sem):
    wid = lax.axis_index("s") * _NC + lax.axis_index("c")
    pltpu.sync_copy(idx_hbm.at[pl.ds(wid * _N_CHUNKS, _N_CHUNKS)], idx_v)
    out_base = wid * _ROWS_PER_W

    def body(j, carry):
        pltpu.async_copy(table_hbm.at[idx_v.at[j]], rows_v, sem).wait()
        pltpu.sync_copy(rows_v, out_hbm.at[pl.ds(out_base + j * _CHUNK, _CHUNK)])
        return carry

    lax.fori_loop(0, _N_CHUNKS, body, 0)


def kernel(token_ids, embeds):
    idx = token_ids.reshape(-1).astype(jnp.int32).reshape(_NW * _N_CHUNKS, _CHUNK)
    out = _gather(idx, embeds)
    return out.reshape(_B0, _B1, _D)


# speedup vs baseline: 1.1151x; 1.1151x over previous
import functools

import jax
import jax.numpy as jnp
from jax import lax
from jax.experimental import pallas as pl
from jax.experimental.pallas import tpu as pltpu
from jax.experimental.pallas import tpu_sc as plsc

_B0, _B1 = 4096, 200
_D = 64
_B_TOTAL = _B0 * _B1
_NC, _NS = 2, 16
_NW = _NC * _NS
_ROWS_PER_W = _B_TOTAL // _NW
_CHUNK = 128
_N_CHUNKS = _ROWS_PER_W // _CHUNK
_NBUF = 8
_LAG = 4
_N_GROUPS = _N_CHUNKS // _NBUF - 1

_mesh = plsc.VectorSubcoreMesh(core_axis_name="c", subcore_axis_name="s")


@functools.partial(
    pl.kernel,
    mesh=_mesh,
    compiler_params=pltpu.CompilerParams(use_tc_tiling_on_sc=False),
    out_type=jax.ShapeDtypeStruct((_B_TOTAL, _D), jnp.float32),
    scratch_types=[
        pltpu.VMEM((_N_CHUNKS, _CHUNK), jnp.int32),
        *[pltpu.VMEM((_CHUNK, _D), jnp.float32) for _ in range(_NBUF)],
        *[pltpu.SemaphoreType.DMA for _ in range(2 * _NBUF)],
    ],
)
def _gather(idx_hbm, table_hbm, out_hbm, idx_v, *scratch):
    bufs = scratch[:_NBUF]
    gsem = scratch[_NBUF:2 * _NBUF]
    ssem = scratch[2 * _NBUF:]

    wid = lax.axis_index("s") * _NC + lax.axis_index("c")
    pltpu.sync_copy(idx_hbm.at[pl.ds(wid * _N_CHUNKS, _N_CHUNKS)], idx_v)
    out_base = wid * _ROWS_PER_W

    def start_gather(j, b):
        pltpu.async_copy(table_hbm.at[idx_v.at[j]], bufs[b], gsem[b])

    def wait_gather(b):
        pltpu.make_async_copy(table_hbm.at[idx_v.at[0]], bufs[b], gsem[b]).wait()

    def start_store(j, b):
        dst = out_hbm.at[pl.ds(out_base + j * _CHUNK, _CHUNK)]
        pltpu.async_copy(bufs[b], dst, ssem[b])

    def wait_store(b):
        dst = out_hbm.at[pl.ds(out_base, _CHUNK)]
        pltpu.make_async_copy(bufs[b], dst, ssem[b]).wait()

    for b in range(_NBUF):
        start_gather(b, b)
    for b in range(_NBUF - _LAG):
        wait_gather(b)
        start_store(b, b)

    def group(g, carry):
        base_j = _NBUF + g * _NBUF
        for b in range(_NBUF):
            j = base_j + b
            wait_store(b)
            start_gather(j, b)
            b2 = (b - _LAG) % _NBUF
            wait_gather(b2)
            start_store(j - _LAG, b2)
        return carry

    lax.fori_loop(0, _N_GROUPS, group, 0)

    for j in range(_N_CHUNKS - _LAG, _N_CHUNKS):
        b = j % _NBUF
        wait_gather(b)
        start_store(j, b)
    for b in range(_NBUF):
        wait_store(b)


def kernel(token_ids, embeds):
    idx = token_ids.reshape(-1).astype(jnp.int32).reshape(_NW * _N_CHUNKS, _CHUNK)
    out = _gather(idx, embeds)
    return out.reshape(_B0, _B1, _D)
